# Initial kernel scaffold; baseline (speedup 1.0000x reference)
import jax
import jax.numpy as jnp
from jax.experimental import pallas as pl


def kernel(x, edge_index, W_l1, b_l1, W_r1, W_l2, b_l2, W_r2):
    raise NotImplementedError("write your pallas kernel here")


# speedup vs baseline: 2.8404x; 2.8404x over previous
import functools

import jax
import jax.numpy as jnp
from jax import lax
from jax.experimental import pallas as pl
from jax.experimental.pallas import tpu as pltpu
from jax.experimental.pallas import tpu_sc as plsc

N_NODES = 10000
D = 128
N_EDGES = 320000

NC = 2
NS = 16
NW = NC * NS
CHUNK = 128
EDGES_PAD = 327680
CHUNKS_PER_W = EDGES_PAD // (NW * CHUNK)
ACC_ROWS = 10112
ACC_CHUNKS = ACC_ROWS // CHUNK


@functools.cache
def _mesh():
    return plsc.VectorSubcoreMesh(core_axis_name="c", subcore_axis_name="s")


def _zero_acc(s, zacc_hbm, rows_v, acc_sp, nk):
    pltpu.sync_copy(zacc_hbm.at[pl.ds(0, CHUNK)], rows_v)

    def zbody(k, carry):
        r0 = (s + k * NS) * CHUNK
        pltpu.sync_copy(rows_v, acc_sp.at[pl.ds(r0, CHUNK)])
        return carry

    lax.fori_loop(0, nk, zbody, 0)


def _flush_acc(c, s, acc_sp, rows_v, outp, nk):
    def fbody(k, carry):
        r0 = (s + k * NS) * CHUNK
        pltpu.sync_copy(acc_sp.at[pl.ds(r0, CHUNK)], rows_v)
        pltpu.sync_copy(rows_v, outp.at[c, pl.ds(r0, CHUNK)])
        return carry

    lax.fori_loop(0, nk, fbody, 0)


def _sc_edge_phase(y, src, dst):
    zacc = jnp.zeros((ACC_ROWS, D), jnp.float32)

    @functools.partial(
        pl.kernel, mesh=_mesh(),
        out_type=jax.ShapeDtypeStruct((NC, ACC_ROWS, D), jnp.float32),
        scratch_types=[
            pltpu.VMEM((CHUNK,), jnp.int32),
            pltpu.VMEM((CHUNK,), jnp.int32),
            pltpu.VMEM((CHUNK, D), jnp.float32),
            pltpu.VMEM_SHARED((ACC_ROWS, D), jnp.float32),
        ])
    def body(y_hbm, src_hbm, dst_hbm, zacc_hbm, outp, src_c, dst_c,
             rows_v, acc_sp):
        c = lax.axis_index("c")
        s = lax.axis_index("s")
        wid = c * NS + s
        nk = (ACC_CHUNKS - 1 - s) // NS + 1

        _zero_acc(s, zacc_hbm, rows_v, acc_sp, nk)
        plsc.subcore_barrier()

        def step(j, carry):
            pltpu.sync_copy(src_hbm.at[wid, j], src_c)
            pltpu.sync_copy(dst_hbm.at[wid, j], dst_c)
            pltpu.sync_copy(y_hbm.at[src_c], rows_v)
            pltpu.sync_copy(rows_v, acc_sp.at[dst_c], add=True)
            return carry

        lax.fori_loop(0, CHUNKS_PER_W, step, 0)
        plsc.subcore_barrier()
        _flush_acc(c, s, acc_sp, rows_v, outp, nk)

    return body(y, src, dst, zacc)


def _sc_count_phase(dst):
    zacc = jnp.zeros((ACC_ROWS, D), jnp.float32)
    ones = jnp.ones((CHUNK, D), jnp.float32)

    @functools.partial(
        pl.kernel, mesh=_mesh(),
        out_type=jax.ShapeDtypeStruct((NC, ACC_ROWS, D), jnp.float32),
        scratch_types=[
            pltpu.VMEM((CHUNK,), jnp.int32),
            pltpu.VMEM((CHUNK, D), jnp.float32),
            pltpu.VMEM((CHUNK, D), jnp.float32),
            pltpu.VMEM_SHARED((ACC_ROWS, D), jnp.float32),
        ])
    def body(dst_hbm, zacc_hbm, ones_hbm, outp, dst_c, ones_v, rows_v,
             acc_sp):
        c = lax.axis_index("c")
        s = lax.axis_index("s")
        wid = c * NS + s
        nk = (ACC_CHUNKS - 1 - s) // NS + 1

        _zero_acc(s, zacc_hbm, rows_v, acc_sp, nk)
        pltpu.sync_copy(ones_hbm, ones_v)
        plsc.subcore_barrier()

        def step(j, carry):
            pltpu.sync_copy(dst_hbm.at[wid, j], dst_c)
            pltpu.sync_copy(ones_v, acc_sp.at[dst_c], add=True)
            return carry

        lax.fori_loop(0, CHUNKS_PER_W, step, 0)
        plsc.subcore_barrier()
        _flush_acc(c, s, acc_sp, rows_v, outp, nk)

    return body(dst, zacc, ones)


BR = 1000
GRID = N_NODES // BR


def _mm2_body(x_ref, wl_ref, wr_ref, y_ref, z_ref):
    xb = x_ref[...]
    y_ref[...] = jnp.dot(xb, wl_ref[...], preferred_element_type=jnp.float32)
    z_ref[...] = jnp.dot(xb, wr_ref[...], preferred_element_type=jnp.float32)


def _tc_mm2(x, wl, wr):
    wspec = pl.BlockSpec((D, D), lambda i: (0, 0))
    return pl.pallas_call(
        _mm2_body,
        grid=(GRID,),
        in_specs=[pl.BlockSpec((BR, D), lambda i: (i, 0)), wspec, wspec],
        out_specs=[pl.BlockSpec((BR, D), lambda i: (i, 0))] * 2,
        out_shape=[jax.ShapeDtypeStruct((N_NODES, D), jnp.float32)] * 2,
    )(x, wl, wr)


def _combine(p_ref, c_ref, z_ref, b_ref):
    agg = p_ref[0] + p_ref[1]
    cnt = c_ref[0, :, :1] + c_ref[1, :, :1]
    inv = 1.0 / jnp.maximum(cnt, 1.0)
    t = agg * inv + z_ref[...] + b_ref[...]
    n = jnp.sqrt(jnp.sum(t * t, axis=-1, keepdims=True))
    return t / jnp.maximum(n, 1e-12)


def _mid_body(p_ref, c_ref, z_ref, b_ref, wl2_ref, wr2_ref, y2_ref, z2_ref):
    h = jnp.maximum(_combine(p_ref, c_ref, z_ref, b_ref), 0.0)
    y2_ref[...] = jnp.dot(h, wl2_ref[...], preferred_element_type=jnp.float32)
    z2_ref[...] = jnp.dot(h, wr2_ref[...], preferred_element_type=jnp.float32)


def _tc_mid(p, cp, z1, b1, wl2, wr2):
    wspec = pl.BlockSpec((D, D), lambda i: (0, 0))
    return pl.pallas_call(
        _mid_body,
        grid=(GRID,),
        in_specs=[
            pl.BlockSpec((NC, BR, D), lambda i: (0, i, 0)),
            pl.BlockSpec((NC, BR, D), lambda i: (0, i, 0)),
            pl.BlockSpec((BR, D), lambda i: (i, 0)),
            pl.BlockSpec((1, D), lambda i: (0, 0)),
            wspec, wspec,
        ],
        out_specs=[pl.BlockSpec((BR, D), lambda i: (i, 0))] * 2,
        out_shape=[jax.ShapeDtypeStruct((N_NODES, D), jnp.float32)] * 2,
    )(p, cp, z1, b1, wl2, wr2)


def _final_body(p_ref, c_ref, z_ref, b_ref, o_ref):
    o_ref[...] = _combine(p_ref, c_ref, z_ref, b_ref)


def _tc_final(q, cp, z2, b2):
    return pl.pallas_call(
        _final_body,
        grid=(GRID,),
        in_specs=[
            pl.BlockSpec((NC, BR, D), lambda i: (0, i, 0)),
            pl.BlockSpec((NC, BR, D), lambda i: (0, i, 0)),
            pl.BlockSpec((BR, D), lambda i: (i, 0)),
            pl.BlockSpec((1, D), lambda i: (0, 0)),
        ],
        out_specs=pl.BlockSpec((BR, D), lambda i: (i, 0)),
        out_shape=jax.ShapeDtypeStruct((N_NODES, D), jnp.float32),
    )(q, cp, z2, b2)


def kernel(x, edge_index, W_l1, b_l1, W_r1, W_l2, b_l2, W_r2):
    e = edge_index.astype(jnp.int32)
    pad = EDGES_PAD - N_EDGES
    src = jnp.concatenate([e[0], jnp.zeros((pad,), jnp.int32)])
    dst = jnp.concatenate([e[1], jnp.full((pad,), N_NODES, jnp.int32)])
    src = src.reshape(NW, CHUNKS_PER_W, CHUNK)
    dst = dst.reshape(NW, CHUNKS_PER_W, CHUNK)

    cp = _sc_count_phase(dst)
    y1, z1 = _tc_mm2(x, W_l1, W_r1)
    p1 = _sc_edge_phase(y1, src, dst)
    y2, z2 = _tc_mid(p1, cp, z1, b_l1.reshape(1, D), W_l2, W_r2)
    q = _sc_edge_phase(y2, src, dst)
    return _tc_final(q, cp, z2, b_l2.reshape(1, D))


# speedup vs baseline: 3.4546x; 1.2162x over previous
import functools

import jax
import jax.numpy as jnp
from jax import lax
from jax.experimental import pallas as pl
from jax.experimental.pallas import tpu as pltpu
from jax.experimental.pallas import tpu_sc as plsc

N_NODES = 10000
D = 128
N_EDGES = 320000

NC = 2
NS = 16
NW = NC * NS
CHUNK = 128
EDGES_PAD = 327680
CHUNKS_PER_W = EDGES_PAD // (NW * CHUNK)
ACC_ROWS = 10112
ACC_CHUNKS = ACC_ROWS // CHUNK
G = 16
NGROUPS = CHUNKS_PER_W // G


@functools.cache
def _mesh():
    return plsc.VectorSubcoreMesh(core_axis_name="c", subcore_axis_name="s")


def _zero_acc(s, zacc_hbm, rows_v, acc_sp, nk):
    pltpu.sync_copy(zacc_hbm.at[pl.ds(0, CHUNK)], rows_v)

    def zbody(k, carry):
        r0 = (s + k * NS) * CHUNK
        pltpu.sync_copy(rows_v, acc_sp.at[pl.ds(r0, CHUNK)])
        return carry

    lax.fori_loop(0, nk, zbody, 0)


def _flush_acc(c, s, acc_sp, rows_v, outp, nk):
    def fbody(k, carry):
        r0 = (s + k * NS) * CHUNK
        pltpu.sync_copy(acc_sp.at[pl.ds(r0, CHUNK)], rows_v)
        pltpu.sync_copy(rows_v, outp.at[c, pl.ds(r0, CHUNK)])
        return carry

    lax.fori_loop(0, nk, fbody, 0)


def _sc_edge_phase(y, src, dst):
    zacc = jnp.zeros((ACC_ROWS, D), jnp.float32)

    @functools.partial(
        pl.kernel, mesh=_mesh(),
        out_type=jax.ShapeDtypeStruct((NC, ACC_ROWS, D), jnp.float32),
        scratch_types=[
            pltpu.VMEM((G, CHUNK), jnp.int32),
            pltpu.VMEM((G, CHUNK), jnp.int32),
            pltpu.VMEM((CHUNK, D), jnp.float32),
            pltpu.VMEM((CHUNK, D), jnp.float32),
            pltpu.VMEM_SHARED((ACC_ROWS, D), jnp.float32),
            pltpu.SemaphoreType.DMA,
            pltpu.SemaphoreType.DMA,
        ])
    def body(y_hbm, src_hbm, dst_hbm, zacc_hbm, outp, src_g, dst_g,
             rows0, rows1, acc_sp, sem0, sem1):
        c = lax.axis_index("c")
        s = lax.axis_index("s")
        wid = c * NS + s
        nk = (ACC_CHUNKS - 1 - s) // NS + 1

        _zero_acc(s, zacc_hbm, rows0, acc_sp, nk)
        plsc.subcore_barrier()

        def gstart(jj, rows, sem):
            pltpu.async_copy(y_hbm.at[src_g.at[jj]], rows, sem)

        def gwait(rows, sem):
            pltpu.make_async_copy(y_hbm.at[pl.ds(0, CHUNK)], rows, sem).wait()

        def group(g, carry):
            pltpu.sync_copy(src_hbm.at[wid, pl.ds(g * G, G)], src_g)
            pltpu.sync_copy(dst_hbm.at[wid, pl.ds(g * G, G)], dst_g)
            gstart(0, rows0, sem0)

            def dbl(k, c2):
                gstart(2 * k + 1, rows1, sem1)
                gwait(rows0, sem0)
                pltpu.sync_copy(rows0, acc_sp.at[dst_g.at[2 * k]], add=True)

                @pl.when(k < G // 2 - 1)
                def _():
                    gstart(2 * k + 2, rows0, sem0)

                gwait(rows1, sem1)
                pltpu.sync_copy(rows1, acc_sp.at[dst_g.at[2 * k + 1]],
                                add=True)
                return c2

            lax.fori_loop(0, G // 2, dbl, 0)
            return carry

        lax.fori_loop(0, NGROUPS, group, 0)
        plsc.subcore_barrier()
        _flush_acc(c, s, acc_sp, rows0, outp, nk)

    return body(y, src, dst, zacc)


def _sc_count_phase(dst):
    zacc = jnp.zeros((ACC_ROWS, D), jnp.float32)
    ones = jnp.ones((CHUNK, D), jnp.float32)

    @functools.partial(
        pl.kernel, mesh=_mesh(),
        out_type=jax.ShapeDtypeStruct((NC, ACC_ROWS, D), jnp.float32),
        scratch_types=[
            pltpu.VMEM((G, CHUNK), jnp.int32),
            pltpu.VMEM((CHUNK, D), jnp.float32),
            pltpu.VMEM((CHUNK, D), jnp.float32),
            pltpu.VMEM_SHARED((ACC_ROWS, D), jnp.float32),
            pltpu.SemaphoreType.DMA,
        ])
    def body(dst_hbm, zacc_hbm, ones_hbm, outp, dst_g, ones_v, rows_v,
             acc_sp, sem):
        c = lax.axis_index("c")
        s = lax.axis_index("s")
        wid = c * NS + s
        nk = (ACC_CHUNKS - 1 - s) // NS + 1

        _zero_acc(s, zacc_hbm, rows_v, acc_sp, nk)
        pltpu.sync_copy(ones_hbm, ones_v)
        plsc.subcore_barrier()

        def group(g, carry):
            pltpu.sync_copy(dst_hbm.at[wid, pl.ds(g * G, G)], dst_g)

            def fire(j, c2):
                pltpu.async_copy(ones_v, acc_sp.at[dst_g.at[j]], sem,
                                 add=True)
                return c2

            lax.fori_loop(0, G, fire, 0)

            def drain(j, c2):
                pltpu.make_async_copy(zacc_hbm.at[pl.ds(0, CHUNK)], rows_v,
                                      sem).wait()
                return c2

            lax.fori_loop(0, G, drain, 0)
            return carry

        lax.fori_loop(0, NGROUPS, group, 0)
        plsc.subcore_barrier()
        _flush_acc(c, s, acc_sp, rows_v, outp, nk)

    return body(dst, zacc, ones)


BR = 1000
GRID = N_NODES // BR


def _mm2_body(x_ref, wl_ref, wr_ref, y_ref, z_ref):
    xb = x_ref[...]
    y_ref[...] = jnp.dot(xb, wl_ref[...], preferred_element_type=jnp.float32)
    z_ref[...] = jnp.dot(xb, wr_ref[...], preferred_element_type=jnp.float32)


def _tc_mm2(x, wl, wr):
    wspec = pl.BlockSpec((D, D), lambda i: (0, 0))
    return pl.pallas_call(
        _mm2_body,
        grid=(GRID,),
        in_specs=[pl.BlockSpec((BR, D), lambda i: (i, 0)), wspec, wspec],
        out_specs=[pl.BlockSpec((BR, D), lambda i: (i, 0))] * 2,
        out_shape=[jax.ShapeDtypeStruct((N_NODES, D), jnp.float32)] * 2,
    )(x, wl, wr)


def _combine(p_ref, c_ref, z_ref, b_ref):
    agg = p_ref[0] + p_ref[1]
    cnt = c_ref[0, :, :1] + c_ref[1, :, :1]
    inv = 1.0 / jnp.maximum(cnt, 1.0)
    t = agg * inv + z_ref[...] + b_ref[...]
    n = jnp.sqrt(jnp.sum(t * t, axis=-1, keepdims=True))
    return t / jnp.maximum(n, 1e-12)


def _mid_body(p_ref, c_ref, z_ref, b_ref, wl2_ref, wr2_ref, y2_ref, z2_ref):
    h = jnp.maximum(_combine(p_ref, c_ref, z_ref, b_ref), 0.0)
    y2_ref[...] = jnp.dot(h, wl2_ref[...], preferred_element_type=jnp.float32)
    z2_ref[...] = jnp.dot(h, wr2_ref[...], preferred_element_type=jnp.float32)


def _tc_mid(p, cp, z1, b1, wl2, wr2):
    wspec = pl.BlockSpec((D, D), lambda i: (0, 0))
    return pl.pallas_call(
        _mid_body,
        grid=(GRID,),
        in_specs=[
            pl.BlockSpec((NC, BR, D), lambda i: (0, i, 0)),
            pl.BlockSpec((NC, BR, D), lambda i: (0, i, 0)),
            pl.BlockSpec((BR, D), lambda i: (i, 0)),
            pl.BlockSpec((1, D), lambda i: (0, 0)),
            wspec, wspec,
        ],
        out_specs=[pl.BlockSpec((BR, D), lambda i: (i, 0))] * 2,
        out_shape=[jax.ShapeDtypeStruct((N_NODES, D), jnp.float32)] * 2,
    )(p, cp, z1, b1, wl2, wr2)


def _final_body(p_ref, c_ref, z_ref, b_ref, o_ref):
    o_ref[...] = _combine(p_ref, c_ref, z_ref, b_ref)


def _tc_final(q, cp, z2, b2):
    return pl.pallas_call(
        _final_body,
        grid=(GRID,),
        in_specs=[
            pl.BlockSpec((NC, BR, D), lambda i: (0, i, 0)),
            pl.BlockSpec((NC, BR, D), lambda i: (0, i, 0)),
            pl.BlockSpec((BR, D), lambda i: (i, 0)),
            pl.BlockSpec((1, D), lambda i: (0, 0)),
        ],
        out_specs=pl.BlockSpec((BR, D), lambda i: (i, 0)),
        out_shape=jax.ShapeDtypeStruct((N_NODES, D), jnp.float32),
    )(q, cp, z2, b2)


def kernel(x, edge_index, W_l1, b_l1, W_r1, W_l2, b_l2, W_r2):
    e = edge_index.astype(jnp.int32)
    pad = EDGES_PAD - N_EDGES
    src = jnp.concatenate([e[0], jnp.zeros((pad,), jnp.int32)])
    dst = jnp.concatenate([e[1], jnp.full((pad,), N_NODES, jnp.int32)])
    src = src.reshape(NW, CHUNKS_PER_W, CHUNK)
    dst = dst.reshape(NW, CHUNKS_PER_W, CHUNK)

    cp = _sc_count_phase(dst)
    y1, z1 = _tc_mm2(x, W_l1, W_r1)
    p1 = _sc_edge_phase(y1, src, dst)
    y2, z2 = _tc_mid(p1, cp, z1, b_l1.reshape(1, D), W_l2, W_r2)
    q = _sc_edge_phase(y2, src, dst)
    return _tc_final(q, cp, z2, b_l2.reshape(1, D))


# speedup vs baseline: 3.6198x; 1.0478x over previous
import functools

import jax
import jax.numpy as jnp
from jax import lax
from jax.experimental import pallas as pl
from jax.experimental.pallas import tpu as pltpu
from jax.experimental.pallas import tpu_sc as plsc

N_NODES = 10000
D = 128
N_EDGES = 320000

NC = 2
NS = 16
NW = NC * NS
CHUNK = 64
EDGES_PAD = 327680
CHUNKS_PER_W = EDGES_PAD // (NW * CHUNK)
NBUF = 4
ACC_ROWS = 10112
ACC_CHUNKS = ACC_ROWS // CHUNK
G = 32
NGROUPS = CHUNKS_PER_W // G


@functools.cache
def _mesh():
    return plsc.VectorSubcoreMesh(core_axis_name="c", subcore_axis_name="s")


def _zero_acc(s, zacc_hbm, rows_v, acc_sp, nk):
    pltpu.sync_copy(zacc_hbm.at[pl.ds(0, CHUNK)], rows_v)

    def zbody(k, carry):
        r0 = (s + k * NS) * CHUNK
        pltpu.sync_copy(rows_v, acc_sp.at[pl.ds(r0, CHUNK)])
        return carry

    lax.fori_loop(0, nk, zbody, 0)


def _flush_acc(c, s, acc_sp, rows_v, outp, nk):
    def fbody(k, carry):
        r0 = (s + k * NS) * CHUNK
        pltpu.sync_copy(acc_sp.at[pl.ds(r0, CHUNK)], rows_v)
        pltpu.sync_copy(rows_v, outp.at[c, pl.ds(r0, CHUNK)])
        return carry

    lax.fori_loop(0, nk, fbody, 0)


def _sc_edge_phase(y, src, dst):
    zacc = jnp.zeros((ACC_ROWS, D), jnp.float32)

    @functools.partial(
        pl.kernel, mesh=_mesh(),
        out_type=jax.ShapeDtypeStruct((NC, ACC_ROWS, D), jnp.float32),
        scratch_types=[
            pltpu.VMEM((G, CHUNK), jnp.int32),
            pltpu.VMEM((G, CHUNK), jnp.int32),
            pltpu.VMEM((NBUF, CHUNK, D), jnp.float32),
            pltpu.VMEM_SHARED((ACC_ROWS, D), jnp.float32),
        ] + [pltpu.SemaphoreType.DMA] * NBUF)
    def body(y_hbm, src_hbm, dst_hbm, zacc_hbm, outp, src_g, dst_g,
             ring, acc_sp, *sems):
        c = lax.axis_index("c")
        s = lax.axis_index("s")
        wid = c * NS + s
        nk = (ACC_CHUNKS - 1 - s) // NS + 1

        _zero_acc(s, zacc_hbm, ring.at[0], acc_sp, nk)
        plsc.subcore_barrier()

        def gstart(jj, b):
            pltpu.async_copy(y_hbm.at[src_g.at[jj]], ring.at[b], sems[b])

        def gwait(b):
            pltpu.make_async_copy(y_hbm.at[pl.ds(0, CHUNK)], ring.at[b],
                                  sems[b]).wait()

        def group(g, carry):
            pltpu.sync_copy(src_hbm.at[wid, pl.ds(g * G, G)], src_g)
            pltpu.sync_copy(dst_hbm.at[wid, pl.ds(g * G, G)], dst_g)
            for b in range(NBUF):
                gstart(b, b)

            def ringstep(m, c2):
                for b in range(NBUF):
                    j = m * NBUF + b
                    gwait(b)
                    pltpu.sync_copy(ring.at[b], acc_sp.at[dst_g.at[j]],
                                    add=True)

                    @pl.when(j + NBUF < G)
                    def _():
                        gstart(j + NBUF, b)
                return c2

            lax.fori_loop(0, G // NBUF, ringstep, 0)
            return carry

        lax.fori_loop(0, NGROUPS, group, 0)
        plsc.subcore_barrier()
        _flush_acc(c, s, acc_sp, ring.at[0], outp, nk)

    return body(y, src, dst, zacc)


def _sc_count_phase(dst):
    zacc = jnp.zeros((ACC_ROWS, D), jnp.float32)
    ones = jnp.ones((CHUNK, D), jnp.float32)

    @functools.partial(
        pl.kernel, mesh=_mesh(),
        out_type=jax.ShapeDtypeStruct((NC, ACC_ROWS, D), jnp.float32),
        scratch_types=[
            pltpu.VMEM((G, CHUNK), jnp.int32),
            pltpu.VMEM((CHUNK, D), jnp.float32),
            pltpu.VMEM((CHUNK, D), jnp.float32),
            pltpu.VMEM_SHARED((ACC_ROWS, D), jnp.float32),
            pltpu.SemaphoreType.DMA,
        ])
    def body(dst_hbm, zacc_hbm, ones_hbm, outp, dst_g, ones_v, rows_v,
             acc_sp, sem):
        c = lax.axis_index("c")
        s = lax.axis_index("s")
        wid = c * NS + s
        nk = (ACC_CHUNKS - 1 - s) // NS + 1

        _zero_acc(s, zacc_hbm, rows_v, acc_sp, nk)
        pltpu.sync_copy(ones_hbm, ones_v)
        plsc.subcore_barrier()

        def group(g, carry):
            pltpu.sync_copy(dst_hbm.at[wid, pl.ds(g * G, G)], dst_g)

            def fire(j, c2):
                pltpu.async_copy(ones_v, acc_sp.at[dst_g.at[j]], sem,
                                 add=True)
                return c2

            lax.fori_loop(0, G, fire, 0)

            def drain(j, c2):
                pltpu.make_async_copy(zacc_hbm.at[pl.ds(0, CHUNK)], rows_v,
                                      sem).wait()
                return c2

            lax.fori_loop(0, G, drain, 0)
            return carry

        lax.fori_loop(0, NGROUPS, group, 0)
        plsc.subcore_barrier()
        _flush_acc(c, s, acc_sp, rows_v, outp, nk)

    return body(dst, zacc, ones)


BR = 1000
GRID = N_NODES // BR


def _mm2_body(x_ref, wl_ref, wr_ref, y_ref, z_ref):
    xb = x_ref[...]
    y_ref[...] = jnp.dot(xb, wl_ref[...], preferred_element_type=jnp.float32)
    z_ref[...] = jnp.dot(xb, wr_ref[...], preferred_element_type=jnp.float32)


def _tc_mm2(x, wl, wr):
    wspec = pl.BlockSpec((D, D), lambda i: (0, 0))
    return pl.pallas_call(
        _mm2_body,
        grid=(GRID,),
        in_specs=[pl.BlockSpec((BR, D), lambda i: (i, 0)), wspec, wspec],
        out_specs=[pl.BlockSpec((BR, D), lambda i: (i, 0))] * 2,
        out_shape=[jax.ShapeDtypeStruct((N_NODES, D), jnp.float32)] * 2,
    )(x, wl, wr)


def _combine(p_ref, c_ref, z_ref, b_ref):
    agg = p_ref[0] + p_ref[1]
    cnt = c_ref[0, :, :1] + c_ref[1, :, :1]
    inv = 1.0 / jnp.maximum(cnt, 1.0)
    t = agg * inv + z_ref[...] + b_ref[...]
    n = jnp.sqrt(jnp.sum(t * t, axis=-1, keepdims=True))
    return t / jnp.maximum(n, 1e-12)


def _mid_body(p_ref, c_ref, z_ref, b_ref, wl2_ref, wr2_ref, y2_ref, z2_ref):
    h = jnp.maximum(_combine(p_ref, c_ref, z_ref, b_ref), 0.0)
    y2_ref[...] = jnp.dot(h, wl2_ref[...], preferred_element_type=jnp.float32)
    z2_ref[...] = jnp.dot(h, wr2_ref[...], preferred_element_type=jnp.float32)


def _tc_mid(p, cp, z1, b1, wl2, wr2):
    wspec = pl.BlockSpec((D, D), lambda i: (0, 0))
    return pl.pallas_call(
        _mid_body,
        grid=(GRID,),
        in_specs=[
            pl.BlockSpec((NC, BR, D), lambda i: (0, i, 0)),
            pl.BlockSpec((NC, BR, D), lambda i: (0, i, 0)),
            pl.BlockSpec((BR, D), lambda i: (i, 0)),
            pl.BlockSpec((1, D), lambda i: (0, 0)),
            wspec, wspec,
        ],
        out_specs=[pl.BlockSpec((BR, D), lambda i: (i, 0))] * 2,
        out_shape=[jax.ShapeDtypeStruct((N_NODES, D), jnp.float32)] * 2,
    )(p, cp, z1, b1, wl2, wr2)


def _final_body(p_ref, c_ref, z_ref, b_ref, o_ref):
    o_ref[...] = _combine(p_ref, c_ref, z_ref, b_ref)


def _tc_final(q, cp, z2, b2):
    return pl.pallas_call(
        _final_body,
        grid=(GRID,),
        in_specs=[
            pl.BlockSpec((NC, BR, D), lambda i: (0, i, 0)),
            pl.BlockSpec((NC, BR, D), lambda i: (0, i, 0)),
            pl.BlockSpec((BR, D), lambda i: (i, 0)),
            pl.BlockSpec((1, D), lambda i: (0, 0)),
        ],
        out_specs=pl.BlockSpec((BR, D), lambda i: (i, 0)),
        out_shape=jax.ShapeDtypeStruct((N_NODES, D), jnp.float32),
    )(q, cp, z2, b2)


def kernel(x, edge_index, W_l1, b_l1, W_r1, W_l2, b_l2, W_r2):
    e = edge_index.astype(jnp.int32)
    pad = EDGES_PAD - N_EDGES
    src = jnp.concatenate([e[0], jnp.zeros((pad,), jnp.int32)])
    dst = jnp.concatenate([e[1], jnp.full((pad,), N_NODES, jnp.int32)])
    src = src.reshape(NW, CHUNKS_PER_W, CHUNK)
    dst = dst.reshape(NW, CHUNKS_PER_W, CHUNK)

    cp = _sc_count_phase(dst)
    y1, z1 = _tc_mm2(x, W_l1, W_r1)
    p1 = _sc_edge_phase(y1, src, dst)
    y2, z2 = _tc_mid(p1, cp, z1, b_l1.reshape(1, D), W_l2, W_r2)
    q = _sc_edge_phase(y2, src, dst)
    return _tc_final(q, cp, z2, b_l2.reshape(1, D))
